# baseline (speedup 1.0000x reference)
import functools

import jax
import jax.numpy as jnp
from jax import lax
from jax.experimental import pallas as pl
from jax.experimental.pallas import tpu as pltpu
from jax.experimental.pallas import tpu_sc as plsc

_NC, _NS, _L = 2, 16, 16
_NW = _NC * _NS
_R = 8
_UNROLL = 8


@functools.lru_cache(maxsize=None)
def _make_sc_kernel(B, S, D):
    rows_per_w = S // _NW
    nchunks = rows_per_w // _R
    ch = _R * D

    mesh = plsc.VectorSubcoreMesh(core_axis_name="c", subcore_axis_name="s")

    @functools.partial(
        pl.kernel,
        out_type=jax.ShapeDtypeStruct((B * S * D,), jnp.float32),
        mesh=mesh,
        scratch_types=[
            pltpu.VMEM((ch,), jnp.float32),
            pltpu.VMEM((ch,), jnp.float32),
        ],
    )
    def sc_add(x_hbm, pos_hbm, out_hbm, pbuf, xbuf):
        wid = lax.axis_index("s") * _NC + lax.axis_index("c")
        base_row = wid * rows_per_w

        def chunk_body(cc, carry):
            row0 = base_row + cc * _R
            pltpu.sync_copy(pos_hbm.at[pl.ds(row0 * D, ch)], pbuf)
            for b in range(B):
                off = (b * S + row0) * D
                pltpu.sync_copy(x_hbm.at[pl.ds(off, ch)], xbuf)

                def add_body(i, c2):
                    base = i * (_L * _UNROLL)
                    for u in range(_UNROLL):
                        sl = pl.ds(base + u * _L, _L)
                        plsc.addupdate(xbuf.at[sl], pbuf[sl])
                    return c2

                lax.fori_loop(0, ch // (_L * _UNROLL), add_body, 0)
                pltpu.sync_copy(xbuf, out_hbm.at[pl.ds(off, ch)])
            return carry

        lax.fori_loop(0, nchunks, chunk_body, 0)

    return sc_add


def kernel(x, pos_embedding):
    B, S, D = x.shape
    sc_add = _make_sc_kernel(B, S, D)
    out = sc_add(x.reshape(-1), pos_embedding[:S].reshape(-1))
    return out.reshape(B, S, D)
